# Initial kernel scaffold; baseline (speedup 1.0000x reference)
import jax
import jax.numpy as jnp
from jax.experimental import pallas as pl


def kernel(hidden_states, W_router, W1, V, W2):
    raise NotImplementedError("write your pallas kernel here")


# speedup vs baseline: 6.5614x; 6.5614x over previous
import functools

import jax
import jax.numpy as jnp
from jax.experimental import pallas as pl
from jax.experimental.pallas import tpu as pltpu

T = 2048
D = 768
FF = 2048
E = 64
K = 2
A = T * K
M = 128
NT = 96
FT = 512
NF = FF // FT
TM = 256

_SQRT1_2 = 0.7071067811865476


def _router_body(x_ref, wr_ref, idx_ref, w_ref):
    x = x_ref[...]
    logits = jax.lax.dot_general(
        x, wr_ref[...], (((1,), (0,)), ((), ())),
        preferred_element_type=jnp.float32)
    m = jnp.max(logits, axis=-1, keepdims=True)
    p = jnp.exp(logits - m)
    p = p / jnp.sum(p, axis=-1, keepdims=True)
    lane = jax.lax.broadcasted_iota(jnp.int32, p.shape, 1)
    v1 = jnp.max(p, axis=-1, keepdims=True)
    i1 = jnp.min(jnp.where(p >= v1, lane, E), axis=-1, keepdims=True)
    p2 = jnp.where(lane == i1, -jnp.inf, p)
    v2 = jnp.max(p2, axis=-1, keepdims=True)
    i2 = jnp.min(jnp.where(p2 >= v2, lane, E), axis=-1, keepdims=True)
    s = v1 + v2
    idx_ref[...] = jnp.concatenate([i1, i2], axis=1)
    w_ref[...] = jnp.concatenate([v1 / s, v2 / s], axis=1)


def _route(x, W_router):
    idx, w = pl.pallas_call(
        _router_body,
        grid=(T // TM,),
        in_specs=[
            pl.BlockSpec((TM, D), lambda t: (t, 0)),
            pl.BlockSpec((D, E), lambda t: (0, 0)),
        ],
        out_specs=[
            pl.BlockSpec((TM, K), lambda t: (t, 0)),
            pl.BlockSpec((TM, K), lambda t: (t, 0)),
        ],
        out_shape=[
            jax.ShapeDtypeStruct((T, K), jnp.int32),
            jax.ShapeDtypeStruct((T, K), jnp.float32),
        ],
    )(x, W_router)
    return idx, w


def _plan(idx, w):
    ids = idx.reshape(A)
    wf = w.reshape(A)
    order = jnp.argsort(ids, stable=True).astype(jnp.int32)
    sorted_e = ids[order]
    sorted_tok = (order // K).astype(jnp.int32)
    sorted_w = wf[order]
    counts = jnp.bincount(ids, length=E).astype(jnp.int32)
    tiles_pe = (counts + M - 1) // M
    tile_cum = jnp.cumsum(tiles_pe)
    tile_start = tile_cum - tiles_pe
    tt = jnp.arange(NT, dtype=jnp.int32)
    te_raw = jnp.clip(
        jnp.searchsorted(tile_cum, tt, side="right"), 0, E - 1).astype(jnp.int32)
    real = tt < tile_cum[-1]
    nvalid = jnp.where(
        real, jnp.clip(counts[te_raw] - (tt - tile_start[te_raw]) * M, 0, M),
        0).astype(jnp.int32)
    last_e = jnp.max(ids).astype(jnp.int32)
    tile_expert = jnp.where(real, te_raw, last_e).astype(jnp.int32)
    cnt_excl = jnp.cumsum(counts) - counts
    rloc = jnp.arange(A, dtype=jnp.int32) - cnt_excl[sorted_e]
    dest = (tile_start[sorted_e] * M + rloc).astype(jnp.int32)
    padded_tok = jnp.zeros(NT * M, jnp.int32).at[dest].set(sorted_tok)
    padded_w = jnp.zeros((NT * M, 1), jnp.float32).at[dest, 0].set(sorted_w)
    return tile_expert, padded_tok, nvalid, padded_w


def _moe_body(te_ref, pt_ref, nv_ref, x_ref, w1_ref, v_ref, w2_ref, wc_ref,
              out_ref, xg, acc):
    t = pl.program_id(0)
    f = pl.program_id(1)
    nv = nv_ref[t]

    @pl.when(jnp.logical_and(t == 0, f == 0))
    def _zero():
        out_ref[...] = jnp.zeros_like(out_ref)

    @pl.when(jnp.logical_and(nv > 0, f == 0))
    def _gather():
        def body(i, carry):
            tok = pt_ref[t * M + i]
            xg[pl.ds(i, 1), :] = x_ref[pl.ds(tok, 1), :]
            return carry
        jax.lax.fori_loop(0, nv, body, 0)

    @pl.when(nv > 0)
    def _compute():
        xv = xg[...]
        h = jnp.dot(xv, w1_ref[0], preferred_element_type=jnp.float32)
        g = h * 0.5 * (1.0 + jax.lax.erf(h * _SQRT1_2))
        hv = jnp.dot(xv, v_ref[0], preferred_element_type=jnp.float32)
        o = jnp.dot(g * hv, w2_ref[0], preferred_element_type=jnp.float32)

        @pl.when(f == 0)
        def _():
            acc[...] = o

        @pl.when(f > 0)
        def _():
            acc[...] += o

    @pl.when(jnp.logical_and(nv > 0, f == NF - 1))
    def _scatter():
        acc[...] = acc[...] * wc_ref[...]

        def body(i, carry):
            tok = pt_ref[t * M + i]
            out_ref[pl.ds(tok, 1), :] = (
                out_ref[pl.ds(tok, 1), :] + acc[pl.ds(i, 1), :])
            return carry
        jax.lax.fori_loop(0, nv, body, 0)


def _w1_map(t, f, te, pt, nv):
    return te[t], 0, jnp.where(nv[t] == 0, NF - 1, f)


def _v_map(t, f, te, pt, nv):
    return te[t], 0, jnp.where(nv[t] == 0, NF - 1, f)


def _w2_map(t, f, te, pt, nv):
    return te[t], jnp.where(nv[t] == 0, NF - 1, f), 0


def _moe(x, W1, V, W2, tile_expert, padded_tok, nvalid, padded_w):
    grid_spec = pltpu.PrefetchScalarGridSpec(
        num_scalar_prefetch=3,
        grid=(NT, NF),
        in_specs=[
            pl.BlockSpec((T, D), lambda t, f, te, pt, nv: (0, 0)),
            pl.BlockSpec((1, D, FT), _w1_map),
            pl.BlockSpec((1, D, FT), _v_map),
            pl.BlockSpec((1, FT, D), _w2_map),
            pl.BlockSpec((M, 1), lambda t, f, te, pt, nv: (t, 0)),
        ],
        out_specs=pl.BlockSpec((T, D), lambda t, f, te, pt, nv: (0, 0)),
        scratch_shapes=[
            pltpu.VMEM((M, D), jnp.float32),
            pltpu.VMEM((M, D), jnp.float32),
        ],
    )
    return pl.pallas_call(
        _moe_body,
        grid_spec=grid_spec,
        out_shape=jax.ShapeDtypeStruct((T, D), jnp.float32),
    )(tile_expert, padded_tok, nvalid, x, W1, V, W2, padded_w)


def kernel(hidden_states, W_router, W1, V, W2):
    b, s, d = hidden_states.shape
    x = hidden_states.reshape(-1, d)
    idx, w = _route(x, W_router)
    tile_expert, padded_tok, nvalid, padded_w = _plan(idx, w)
    out = _moe(x, W1, V, W2, tile_expert, padded_tok, nvalid, padded_w)
    return out.reshape(b, s, d)
